# baseline (speedup 1.0000x reference)
import functools

import jax
import jax.numpy as jnp
from jax import lax
from jax.experimental import pallas as pl
from jax.experimental.pallas import tpu as pltpu
from jax.experimental.pallas import tpu_sc as plsc

_N = 10000
_E = 320000
_D = 128
_H = 64
_HP = 128
_B = 64

_NC = 2
_NS = 16
_NW = _NC * _NS
_EPW = _E // _NW
_CHUNK = 80
_NCHUNK = _EPW // _CHUNK
_NSETS = 4
_PRE = 2
_NSCAT = _NSETS - _PRE
_ISETS = 8
_IPRE = 4
_RPS = 632
_RPS_LAST = _N - _RPS * (_NS - 1)


def _sc_scatter_partials(y, ei, zeros):
  mesh = plsc.VectorSubcoreMesh(core_axis_name="c", subcore_axis_name="s")

  @functools.partial(
      pl.kernel,
      out_type=jax.ShapeDtypeStruct((_NC, _N, _H), jnp.float32),
      mesh=mesh,
      compiler_params=pltpu.CompilerParams(use_tc_tiling_on_sc=False),
      scratch_types=[
          pltpu.VMEM((_ISETS, 2, _CHUNK), jnp.int32),
          pltpu.VMEM((_NSETS, _CHUNK, _H), jnp.float32),
          pltpu.VMEM_SHARED((_N, _H), jnp.float32),
          pltpu.SemaphoreType.DMA((_ISETS,)),
          pltpu.SemaphoreType.DMA((_NSETS,)),
          pltpu.SemaphoreType.DMA((_NSETS,)),
      ],
  )
  def body(y_hbm, ei_hbm, zeros_hbm, out_hbm, idx_ring,
           rows, agg_sh, isem, gsem, ssem):
    c = lax.axis_index("c")
    s = lax.axis_index("s")
    w = c * _NS + s
    base = w * _NCHUNK
    @pl.when(s < _NS - 1)
    def _():
      pltpu.sync_copy(zeros_hbm.at[pl.ds(s * _RPS, _RPS)],
                      agg_sh.at[pl.ds(s * _RPS, _RPS)])

    @pl.when(s == _NS - 1)
    def _():
      pltpu.sync_copy(zeros_hbm.at[pl.ds((_NS - 1) * _RPS, _RPS_LAST)],
                      agg_sh.at[pl.ds((_NS - 1) * _RPS, _RPS_LAST)])

    plsc.subcore_barrier()

    def fire_idx(k):
      ib = lax.rem(k, _ISETS)
      pltpu.async_copy(ei_hbm.at[base + k], idx_ring.at[ib], isem.at[ib])

    def iwait(k):
      ib = lax.rem(k, _ISETS)
      pltpu.make_async_copy(ei_hbm.at[0], idx_ring.at[0],
                            isem.at[ib]).wait()

    def fire_gather(k):
      pltpu.async_copy(y_hbm.at[idx_ring.at[lax.rem(k, _ISETS)].at[0]],
                       rows.at[lax.rem(k, _NSETS)], gsem.at[lax.rem(k, _NSETS)])

    def gwait(k):
      b = lax.rem(k, _NSETS)
      pltpu.make_async_copy(y_hbm.at[pl.ds(0, _CHUNK)], rows.at[0],
                            gsem.at[b]).wait()

    def fire_scatter(k):
      b = lax.rem(k, _NSETS)
      pltpu.async_copy(rows.at[b],
                       agg_sh.at[idx_ring.at[lax.rem(k, _ISETS)].at[1]],
                       ssem.at[b], add=True)

    def swait(k):
      b = lax.rem(k, _NSETS)
      pltpu.make_async_copy(y_hbm.at[pl.ds(0, _CHUNK)], rows.at[0],
                            ssem.at[b]).wait()

    for k in range(_IPRE):
      fire_idx(k)
    for k in range(_PRE):
      iwait(k)
      fire_gather(k)

    def step(j, carry):
      gwait(j)

      @pl.when(j >= _NSCAT)
      def _():
        swait(j - _NSCAT)

      @pl.when(j < _NCHUNK - _PRE)
      def _():
        iwait(j + _PRE)
        fire_gather(j + _PRE)

      @pl.when(j < _NCHUNK - _IPRE)
      def _():
        fire_idx(j + _IPRE)

      fire_scatter(j)
      return carry

    lax.fori_loop(0, _NCHUNK, step, 0)
    for k in range(_NSCAT):
      swait(_NCHUNK - _NSCAT + k)
    plsc.subcore_barrier()

    @pl.when(s < _NS - 1)
    def _():
      pltpu.sync_copy(agg_sh.at[pl.ds(s * _RPS, _RPS)],
                      out_hbm.at[c].at[pl.ds(s * _RPS, _RPS)])

    @pl.when(s == _NS - 1)
    def _():
      pltpu.sync_copy(agg_sh.at[pl.ds((_NS - 1) * _RPS, _RPS_LAST)],
                      out_hbm.at[c].at[pl.ds((_NS - 1) * _RPS, _RPS_LAST)])

  return body(y, ei, zeros)


def _mm(x, W):
  def body(x_ref, w_ref, o_ref):
    o_ref[...] = jnp.dot(x_ref[...], w_ref[...],
                         preferred_element_type=jnp.float32)

  return pl.pallas_call(
      body,
      out_shape=jax.ShapeDtypeStruct((x.shape[0], W.shape[1]), jnp.float32),
  )(x, W)


def _combine_mm(y, aggs, W, b2d, scale):
  def body(y_ref, a_ref, w_ref, b_ref, s_ref, o_ref):
    h = s_ref[0, 0] * y_ref[...] + a_ref[0] + a_ref[1] + b_ref[...]
    h = jnp.maximum(h, 0.0)
    o_ref[...] = jnp.dot(h, w_ref[...], preferred_element_type=jnp.float32)

  return pl.pallas_call(
      body,
      out_shape=jax.ShapeDtypeStruct((y.shape[0], W.shape[1]), jnp.float32),
  )(y, aggs, W, b2d, scale)


def _final(y3, aggs, b3_2d, scale3, batch2d, Wf1, bf1_2d, Wf2, bf2_2d):
  def body(y_ref, a_ref, b_ref, s_ref, batch_ref, wf1_ref, bf1_ref, wf2_ref,
           bf2_ref, o_ref):
    h = s_ref[0, 0] * y_ref[...] + a_ref[0] + a_ref[1] + b_ref[...]
    h = jnp.maximum(h, 0.0)
    seg = batch_ref[...]
    mask = (lax.broadcasted_iota(jnp.int32, (_B, _N), 0) == seg
            ).astype(jnp.float32)
    sums = jnp.dot(mask, h, preferred_element_type=jnp.float32)
    counts = jnp.sum(mask, axis=1, keepdims=True)
    pooled = sums / jnp.maximum(counts, 1.0)
    g = jnp.dot(pooled, wf1_ref[...], preferred_element_type=jnp.float32)
    g = jnp.maximum(g + bf1_ref[...], 0.0)
    o_ref[...] = jnp.dot(g, wf2_ref[...],
                         preferred_element_type=jnp.float32) + bf2_ref[...]

  return pl.pallas_call(
      body,
      out_shape=jax.ShapeDtypeStruct((_B, 1), jnp.float32),
  )(y3, aggs, b3_2d, scale3, batch2d, Wf1, bf1_2d, Wf2, bf2_2d)


def kernel(x, edge_index, batch, W1, b1, W2, b2, W3, b3, Wf1, bf1, Wf2, bf2,
           eps1, eps2, eps3):
  ei = jnp.stack([jnp.reshape(edge_index[0], (_NW * _NCHUNK, _CHUNK)),
                  jnp.reshape(edge_index[1], (_NW * _NCHUNK, _CHUNK))], axis=1)
  zeros = jnp.zeros((_N, _H), jnp.float32)
  s1 = jnp.reshape(1.0 + eps1, (1, 1))
  s2 = jnp.reshape(1.0 + eps2, (1, 1))
  s3 = jnp.reshape(1.0 + eps3, (1, 1))
  y1 = _mm(x, W1)
  agg1 = _sc_scatter_partials(y1, ei, zeros)
  y2 = _combine_mm(y1, agg1, W2, jnp.reshape(b1, (1, _H)), s1)
  agg2 = _sc_scatter_partials(y2, ei, zeros)
  y3 = _combine_mm(y2, agg2, W3, jnp.reshape(b2, (1, _H)), s2)
  agg3 = _sc_scatter_partials(y3, ei, zeros)
  out = _final(y3, agg3, jnp.reshape(b3, (1, _H)), s3,
               jnp.reshape(batch, (1, _N)), Wf1, jnp.reshape(bf1, (1, 10)),
               Wf2, jnp.reshape(bf2, (1, 1)))
  return out
